# baseline (speedup 1.0000x reference)
import functools

import jax
import jax.numpy as jnp
from jax import lax
from jax.experimental import pallas as pl
from jax.experimental.pallas import tpu as pltpu
from jax.experimental.pallas import tpu_sc as plsc

S, B, D = 2048, 4, 2048
BS = 256
N = S // BS


@functools.partial(
    pl.kernel,
    mesh=plsc.VectorSubcoreMesh(core_axis_name="c", subcore_axis_name="s"),
    out_type=jax.ShapeDtypeStruct((B, D), jnp.float32),
    scratch_types=[
        pltpu.VMEM((B,), jnp.int32),
        pltpu.VMEM((B, D), jnp.float32),
        pltpu.SemaphoreType.DMA,
    ],
    cost_estimate=pl.CostEstimate(
        flops=0, bytes_accessed=2 * B * D * 4, transcendentals=0),
)
def _sc_embed(t_hbm, table_hbm, emb_hbm, idx_v, rows_v, gsem):
    wid = lax.axis_index("s") * 2 + lax.axis_index("c")

    @pl.when(wid == 0)
    def _gather():
        pltpu.sync_copy(t_hbm, idx_v)
        pltpu.async_copy(table_hbm.at[idx_v], rows_v, gsem).wait()
        pltpu.sync_copy(rows_v, emb_hbm)


def _tc_copy_body(x_ref, out_ref):
    i = pl.program_id(0)

    @pl.when(i < N)
    def _copy():
        out_ref[...] = x_ref[...]


def _patch_body(emb_ref, big_ref, out_ref, sem):
    cp = pltpu.make_async_copy(emb_ref, out_ref.at[S], sem)
    cp.start()
    cp.wait()


def kernel(x, t, table):
    t_emb = _sc_embed(t, table)
    big = pl.pallas_call(
        _tc_copy_body,
        grid=(N + 1,),
        out_shape=jax.ShapeDtypeStruct((S + 1, B, D), x.dtype),
        in_specs=[
            pl.BlockSpec((BS, B, D), lambda i: (jnp.minimum(i, N - 1), 0, 0)),
        ],
        out_specs=pl.BlockSpec((BS, B, D), lambda i: (i, 0, 0)),
        cost_estimate=pl.CostEstimate(
            flops=0, bytes_accessed=2 * S * B * D * 4, transcendentals=0),
    )(x)
    return pl.pallas_call(
        _patch_body,
        out_shape=jax.ShapeDtypeStruct((S + 1, B, D), x.dtype),
        in_specs=[
            pl.BlockSpec(memory_space=pl.ANY),
            pl.BlockSpec(memory_space=pl.ANY),
        ],
        out_specs=pl.BlockSpec(memory_space=pl.ANY),
        scratch_shapes=[pltpu.SemaphoreType.DMA],
        input_output_aliases={1: 0},
        cost_estimate=pl.CostEstimate(
            flops=0, bytes_accessed=2 * B * D * 4, transcendentals=0),
    )(t_emb, big)


# speedup vs baseline: 1.3656x; 1.3656x over previous
import jax
import jax.numpy as jnp
from jax.experimental import pallas as pl
from jax.experimental.pallas import tpu as pltpu

S, B, D = 2048, 4, 2048
CH = 256
NCH = S // CH


def _concat_embed_body(t_ref, x_ref, table_ref, out_ref,
                       buf, in_sems, out_sems, gat_sems):
    gathers = []
    for b in range(B):
        g = pltpu.make_async_copy(
            table_ref.at[t_ref[b]], out_ref.at[S, b], gat_sems.at[b])
        g.start()
        gathers.append(g)

    def in_cp(k):
        return pltpu.make_async_copy(
            x_ref.at[pl.ds(k * CH, CH)], buf.at[k % 2], in_sems.at[k % 2])

    def out_cp(k):
        return pltpu.make_async_copy(
            buf.at[k % 2], out_ref.at[pl.ds(k * CH, CH)], out_sems.at[k % 2])

    in_cp(0).start()
    for k in range(NCH):
        if k + 1 < NCH:
            if k >= 1:
                out_cp(k - 1).wait()
            in_cp(k + 1).start()
        in_cp(k).wait()
        out_cp(k).start()
    if NCH >= 2:
        out_cp(NCH - 2).wait()
    out_cp(NCH - 1).wait()
    for g in gathers:
        g.wait()


def kernel(x, t, table):
    return pl.pallas_call(
        _concat_embed_body,
        out_shape=jax.ShapeDtypeStruct((S + 1, B, D), x.dtype),
        in_specs=[
            pl.BlockSpec(memory_space=pltpu.SMEM),
            pl.BlockSpec(memory_space=pl.ANY),
            pl.BlockSpec(memory_space=pl.ANY),
        ],
        out_specs=pl.BlockSpec(memory_space=pl.ANY),
        scratch_shapes=[
            pltpu.VMEM((2, CH, B, D), jnp.float32),
            pltpu.SemaphoreType.DMA((2,)),
            pltpu.SemaphoreType.DMA((2,)),
            pltpu.SemaphoreType.DMA((B,)),
        ],
    )(t, x, table)
